# baseline (speedup 1.0000x reference)
import functools

import jax
import jax.numpy as jnp
import numpy as np
from jax import lax
from jax.experimental import pallas as pl
from jax.experimental.pallas import tpu as pltpu
from jax.experimental.pallas import tpu_sc as plsc

_B = 16384
_NC, _NS = 2, 16
_NW = _NC * _NS
_BPW = _B // _NW
_NT = 100000
_TDR = _NT // 16
_RPT = 390
_RLAST = _TDR - 15 * _RPT
_LOG2PI = float(np.log(2.0 * np.pi))
_LN2 = 0.6931471805599453


def _sc_body(act_hbm, td16_hbm, mu16_hbm, ls16_hbm, out_hbm,
             act_v, tdv_v, idxt_v, idxm_v,
             bufm_v, bufl_v, buft_v, out_v, tmp_v, red_v, shared_v,
             sem_a, sem_b, sem_c):
    cid = lax.axis_index("c")
    sid = lax.axis_index("s")
    wid = cid * _NS + sid
    base = wid * _BPW
    i16 = lax.iota(jnp.int32, 16)
    f32 = jnp.float32

    is_last = sid == _NS - 1
    row0 = sid * _RPT

    @pl.when(jnp.logical_not(is_last))
    def _():
        pltpu.async_copy(td16_hbm.at[pl.ds(row0, _RPT)],
                         tdv_v.at[pl.ds(0, _RPT)], sem_a).wait()

    @pl.when(is_last)
    def _():
        pltpu.async_copy(td16_hbm.at[pl.ds(15 * _RPT, _RLAST)],
                         tdv_v, sem_a).wait()

    cp_act = pltpu.async_copy(act_hbm.at[pl.ds(base * 3, _BPW * 3)],
                              act_v, sem_b)

    cp_act.wait()

    def pidx(i, carry):
        rows = i16 + 16 * i
        ti = plsc.load_gather(act_v, [rows * 3]).astype(jnp.int32)
        plsc.store_scatter(idxt_v, [rows], lax.shift_right_logical(ti, 4))
        plsc.store_scatter(idxm_v, [rows], lax.shift_right_logical(ti, 3))
        return carry

    lax.fori_loop(0, _BPW // 16, pidx, 0)
    g1 = pltpu.async_copy(td16_hbm.at[idxt_v], buft_v, sem_c)
    g2 = pltpu.async_copy(mu16_hbm.at[idxm_v], bufm_v, sem_c)
    g3 = pltpu.async_copy(ls16_hbm.at[idxm_v], bufl_v, sem_c)

    nrows = jnp.where(is_last, _RLAST, _RPT)

    def p1(j, m):
        x = plsc.load_gather(tdv_v, [jnp.broadcast_to(j, (16,)), i16])
        return jnp.maximum(m, x)

    m16 = lax.fori_loop(0, nrows, p1, jnp.full((16,), -jnp.inf, f32))
    mt = jnp.max(m16)
    tmp_v[...] = jnp.broadcast_to(mt, (16,))
    pltpu.sync_copy(tmp_v, shared_v.at[pl.ds(16 * sid, 16)])
    plsc.subcore_barrier()
    pltpu.sync_copy(shared_v.at[pl.ds(0, 256)], red_v)
    gmax = jnp.max(plsc.load_gather(red_v, [i16 * 16]))

    def p2(j, s):
        x = plsc.load_gather(tdv_v, [jnp.broadcast_to(j, (16,)), i16])
        return s + jnp.exp(x - gmax)

    s16 = lax.fori_loop(0, nrows, p2, jnp.zeros((16,), f32))
    st = jnp.sum(s16)
    tmp_v[...] = jnp.broadcast_to(st, (16,))
    pltpu.sync_copy(tmp_v, shared_v.at[pl.ds(256 + 16 * sid, 16)])
    plsc.subcore_barrier()
    pltpu.sync_copy(shared_v.at[pl.ds(256, 256)], red_v)
    s_tot = jnp.sum(plsc.load_gather(red_v, [i16 * 16]))

    sv = jnp.broadcast_to(s_tot, (16,))
    eb = jnp.right_shift(plsc.bitcast(sv, jnp.int32), 23) & 255
    y = (eb.astype(f32) - 126.5) * _LN2

    def pn(j, yy):
        return yy + sv * jnp.exp(-yy) - 1.0

    y = lax.fori_loop(0, 6, pn, y)
    logz = gmax + y

    g1.wait()
    g2.wait()
    g3.wait()

    def pc(i, carry):
        rows = i16 + 16 * i
        tf = plsc.load_gather(act_v, [rows * 3])
        px = plsc.load_gather(act_v, [rows * 3 + 1])
        py = plsc.load_gather(act_v, [rows * 3 + 2])
        ti = tf.astype(jnp.int32)
        cm = lax.shift_left(ti & 7, 1)
        ct = ti & 15
        mx = plsc.load_gather(bufm_v, [rows, cm])
        my = plsc.load_gather(bufm_v, [rows, cm + 1])
        lx = plsc.load_gather(bufl_v, [rows, cm])
        ly = plsc.load_gather(bufl_v, [rows, cm + 1])
        tg = plsc.load_gather(buft_v, [rows, ct])
        dx = px - mx
        dy = py - my
        q = dx * dx * jnp.exp(-lx) + dy * dy * jnp.exp(-ly)
        res = tg - logz - 0.5 * q - 0.5 * (lx + ly) - _LOG2PI
        plsc.store_scatter(out_v, [rows], res)
        return carry

    lax.fori_loop(0, _BPW // 16, pc, 0)
    pltpu.sync_copy(out_v, out_hbm.at[pl.ds(base, _BPW)])


@functools.cache
def _sc_kernel():
    return pl.kernel(
        _sc_body,
        out_type=jax.ShapeDtypeStruct((_B,), jnp.float32),
        mesh=plsc.VectorSubcoreMesh(core_axis_name="c", subcore_axis_name="s",
                                    num_cores=_NC, num_subcores=_NS),
        scratch_types=[
            pltpu.VMEM((_BPW * 3,), jnp.float32),
            pltpu.VMEM((_RLAST, 16), jnp.float32),
            pltpu.VMEM((_BPW,), jnp.int32),
            pltpu.VMEM((_BPW,), jnp.int32),
            pltpu.VMEM((_BPW, 16), jnp.float32),
            pltpu.VMEM((_BPW, 16), jnp.float32),
            pltpu.VMEM((_BPW, 16), jnp.float32),
            pltpu.VMEM((_BPW,), jnp.float32),
            pltpu.VMEM((16,), jnp.float32),
            pltpu.VMEM((256,), jnp.float32),
            pltpu.VMEM_SHARED((512,), jnp.float32),
            pltpu.SemaphoreType.DMA,
            pltpu.SemaphoreType.DMA,
            pltpu.SemaphoreType.DMA,
        ],
        compiler_params=pltpu.CompilerParams(use_tc_tiling_on_sc=False,
                                             needs_layout_passes=False),
    )


def kernel(action, tool_distribution, log_std, means):
    return _sc_kernel()(
        action.reshape(-1),
        tool_distribution.reshape(-1, 16),
        means.reshape(-1, 16),
        log_std.reshape(-1, 16),
    )


# speedup vs baseline: 3.5901x; 3.5901x over previous
import functools

import jax
import jax.numpy as jnp
import numpy as np
from jax import lax
from jax.experimental import pallas as pl
from jax.experimental.pallas import tpu as pltpu
from jax.experimental.pallas import tpu_sc as plsc

_B = 16384
_NC, _NS = 2, 16
_NW = _NC * _NS
_BPW = _B // _NW
_NT = 100000
_TDR = _NT // 16
_RPT = 390
_RLAST = _TDR - 15 * _RPT
_LOG2PI = float(np.log(2.0 * np.pi))
_LN2 = 0.6931471805599453


def _sc_body(act_hbm, td16_hbm, m0_hbm, m1_hbm, l0_hbm, l1_hbm, out_hbm,
             act_v, tdv_v, idxt_v,
             bm0_v, bm1_v, bl0_v, bl1_v, buft_v, out_v, tmp_v, red_v,
             shared_v, sem_a, sem_b, sem_c):
    cid = lax.axis_index("c")
    sid = lax.axis_index("s")
    wid = cid * _NS + sid
    base = wid * _BPW
    i16 = lax.iota(jnp.int32, 16)
    f32 = jnp.float32

    is_last = sid == _NS - 1
    row0 = sid * _RPT

    @pl.when(jnp.logical_not(is_last))
    def _():
        pltpu.async_copy(td16_hbm.at[pl.ds(row0, _RPT)],
                         tdv_v.at[pl.ds(0, _RPT)], sem_a).wait()

    @pl.when(is_last)
    def _():
        pltpu.async_copy(td16_hbm.at[pl.ds(15 * _RPT, _RLAST)],
                         tdv_v, sem_a).wait()

    cp_act = pltpu.async_copy(act_hbm.at[pl.ds(base * 3, _BPW * 3)],
                              act_v, sem_b)

    cp_act.wait()

    def pidx(i, carry):
        rows = i16 + 16 * i
        ti = plsc.load_gather(act_v, [rows * 3]).astype(jnp.int32)
        plsc.store_scatter(idxt_v, [rows], lax.shift_right_logical(ti, 4))
        return carry

    lax.fori_loop(0, _BPW // 16, pidx, 0)
    g1 = pltpu.async_copy(td16_hbm.at[idxt_v], buft_v, sem_c)
    g2 = pltpu.async_copy(m0_hbm.at[idxt_v], bm0_v, sem_c)
    g3 = pltpu.async_copy(m1_hbm.at[idxt_v], bm1_v, sem_c)
    g4 = pltpu.async_copy(l0_hbm.at[idxt_v], bl0_v, sem_c)
    g5 = pltpu.async_copy(l1_hbm.at[idxt_v], bl1_v, sem_c)

    nrows = jnp.where(is_last, _RLAST, _RPT)

    def p1(j, m):
        x = plsc.load_gather(tdv_v, [jnp.broadcast_to(j, (16,)), i16])
        return jnp.maximum(m, x)

    m16 = lax.fori_loop(0, nrows, p1, jnp.full((16,), -jnp.inf, f32))
    mt = jnp.max(m16)
    tmp_v[...] = jnp.broadcast_to(mt, (16,))
    pltpu.sync_copy(tmp_v, shared_v.at[pl.ds(16 * sid, 16)])
    plsc.subcore_barrier()
    pltpu.sync_copy(shared_v.at[pl.ds(0, 256)], red_v)
    gmax = jnp.max(plsc.load_gather(red_v, [i16 * 16]))

    def p2(j, s):
        x = plsc.load_gather(tdv_v, [jnp.broadcast_to(j, (16,)), i16])
        return s + jnp.exp(x - gmax)

    s16 = lax.fori_loop(0, nrows, p2, jnp.zeros((16,), f32))
    st = jnp.sum(s16)
    tmp_v[...] = jnp.broadcast_to(st, (16,))
    pltpu.sync_copy(tmp_v, shared_v.at[pl.ds(256 + 16 * sid, 16)])
    plsc.subcore_barrier()
    pltpu.sync_copy(shared_v.at[pl.ds(256, 256)], red_v)
    s_tot = jnp.sum(plsc.load_gather(red_v, [i16 * 16]))

    sv = jnp.broadcast_to(s_tot, (16,))
    eb = jnp.right_shift(plsc.bitcast(sv, jnp.int32), 23) & 255
    y = (eb.astype(f32) - 126.5) * _LN2

    def pn(j, yy):
        return yy + sv * jnp.exp(-yy) - 1.0

    y = lax.fori_loop(0, 6, pn, y)
    logz = gmax + y

    g1.wait()
    g2.wait()
    g3.wait()
    g4.wait()
    g5.wait()

    def pc(i, carry):
        rows = i16 + 16 * i
        tf = plsc.load_gather(act_v, [rows * 3])
        px = plsc.load_gather(act_v, [rows * 3 + 1])
        py = plsc.load_gather(act_v, [rows * 3 + 2])
        ti = tf.astype(jnp.int32)
        ct = ti & 15
        mx = plsc.load_gather(bm0_v, [rows, ct])
        my = plsc.load_gather(bm1_v, [rows, ct])
        lx = plsc.load_gather(bl0_v, [rows, ct])
        ly = plsc.load_gather(bl1_v, [rows, ct])
        tg = plsc.load_gather(buft_v, [rows, ct])
        dx = px - mx
        dy = py - my
        q = dx * dx * jnp.exp(-lx) + dy * dy * jnp.exp(-ly)
        res = tg - logz - 0.5 * q - 0.5 * (lx + ly) - _LOG2PI
        plsc.store_scatter(out_v, [rows], res)
        return carry

    lax.fori_loop(0, _BPW // 16, pc, 0)
    pltpu.sync_copy(out_v, out_hbm.at[pl.ds(base, _BPW)])


@functools.cache
def _sc_kernel():
    return pl.kernel(
        _sc_body,
        out_type=jax.ShapeDtypeStruct((_B,), jnp.float32),
        mesh=plsc.VectorSubcoreMesh(core_axis_name="c", subcore_axis_name="s",
                                    num_cores=_NC, num_subcores=_NS),
        scratch_types=[
            pltpu.VMEM((_BPW * 3,), jnp.float32),
            pltpu.VMEM((_RLAST, 16), jnp.float32),
            pltpu.VMEM((_BPW,), jnp.int32),
            pltpu.VMEM((_BPW, 16), jnp.float32),
            pltpu.VMEM((_BPW, 16), jnp.float32),
            pltpu.VMEM((_BPW, 16), jnp.float32),
            pltpu.VMEM((_BPW, 16), jnp.float32),
            pltpu.VMEM((_BPW, 16), jnp.float32),
            pltpu.VMEM((_BPW,), jnp.float32),
            pltpu.VMEM((16,), jnp.float32),
            pltpu.VMEM((256,), jnp.float32),
            pltpu.VMEM_SHARED((512,), jnp.float32),
            pltpu.SemaphoreType.DMA,
            pltpu.SemaphoreType.DMA,
            pltpu.SemaphoreType.DMA,
        ],
        compiler_params=pltpu.CompilerParams(use_tc_tiling_on_sc=False,
                                             needs_layout_passes=False),
    )


def kernel(action, tool_distribution, log_std, means):
    return _sc_kernel()(
        action.reshape(-1),
        tool_distribution.reshape(-1, 16),
        means[:, 0].reshape(-1, 16),
        means[:, 1].reshape(-1, 16),
        log_std[:, 0].reshape(-1, 16),
        log_std[:, 1].reshape(-1, 16),
    )
